# baseline (speedup 1.0000x reference)
import functools

import jax
import jax.numpy as jnp
from jax import lax
from jax.experimental import pallas as pl
from jax.experimental.pallas import tpu as pltpu
from jax.experimental.pallas import tpu_sc as plsc

N = 10000
E = 320000
D = 128
L = 3

NC = 2
NS = 16
NW = NC * NS
EDGES_PER_WORKER = E // NW
CHUNK = 64
CHUNKS_PER_WORKER = 160
PAD = CHUNKS_PER_WORKER * CHUNK - EDGES_PER_WORKER
DUMP = 8
ROWS_PER_SUB = 624
TAIL_ROW0 = 16 * ROWS_PER_SUB
TAIL_ROWS = N + DUMP - TAIL_ROW0


def _spmm_sc(h, idx2, zeros):
    mesh = plsc.VectorSubcoreMesh(core_axis_name="c", subcore_axis_name="s")

    @functools.partial(
        pl.kernel,
        out_type=jax.ShapeDtypeStruct((NC * N, D), jnp.float32),
        mesh=mesh,
        scratch_types=[
            pltpu.VMEM((CHUNKS_PER_WORKER, 2 * CHUNK), jnp.int32),
            pltpu.VMEM((CHUNK, D), jnp.float32),
            pltpu.VMEM((CHUNK, D), jnp.float32),
            pltpu.VMEM_SHARED((N + DUMP, D), jnp.float32),
            pltpu.SemaphoreType.DMA,
            pltpu.SemaphoreType.DMA,
            pltpu.SemaphoreType.DMA,
            pltpu.SemaphoreType.DMA,
            pltpu.SemaphoreType.DMA,
        ],
    )
    def spmm_kernel(h_hbm, idx_hbm, z_hbm, out_hbm,
                    combo_v, rows0, rows1, pooled_sh,
                    sem0, sem1, ssem0, ssem1, zsem):
        c = lax.axis_index("c")
        s = lax.axis_index("s")
        wid = s * NC + c

        row0 = s * ROWS_PER_SUB
        zcp = pltpu.async_copy(z_hbm.at[pl.ds(row0, ROWS_PER_SUB)],
                               pooled_sh.at[pl.ds(row0, ROWS_PER_SUB)], zsem)

        @pl.when(s == NS - 1)
        def _():
            pltpu.async_copy(z_hbm.at[pl.ds(TAIL_ROW0, TAIL_ROWS)],
                             pooled_sh.at[pl.ds(TAIL_ROW0, TAIL_ROWS)],
                             zsem).wait()

        cbase = wid * CHUNKS_PER_WORKER
        pltpu.sync_copy(idx_hbm.at[pl.ds(cbase, CHUNKS_PER_WORKER)], combo_v)
        zcp.wait()
        plsc.subcore_barrier()

        pltpu.async_copy(h_hbm.at[combo_v.at[0, pl.ds(0, CHUNK)]], rows0, sem0)
        pltpu.async_copy(h_hbm.at[combo_v.at[1, pl.ds(0, CHUNK)]], rows1, sem1)

        @pl.loop(0, CHUNKS_PER_WORKER - 2, step=2)
        def _(k):
            pltpu.make_async_copy(h_hbm.at[combo_v.at[k, pl.ds(0, CHUNK)]],
                                  rows0, sem0).wait()
            pltpu.async_copy(rows0,
                             pooled_sh.at[combo_v.at[k, pl.ds(CHUNK, CHUNK)]],
                             ssem0, add=True)
            pltpu.make_async_copy(h_hbm.at[combo_v.at[k + 1, pl.ds(0, CHUNK)]],
                                  rows1, sem1).wait()
            pltpu.async_copy(rows1,
                             pooled_sh.at[combo_v.at[k + 1,
                                                     pl.ds(CHUNK, CHUNK)]],
                             ssem1, add=True)
            pltpu.make_async_copy(
                rows0, pooled_sh.at[combo_v.at[k, pl.ds(CHUNK, CHUNK)]],
                ssem0).wait()
            pltpu.async_copy(h_hbm.at[combo_v.at[k + 2, pl.ds(0, CHUNK)]],
                             rows0, sem0)
            pltpu.make_async_copy(
                rows1, pooled_sh.at[combo_v.at[k + 1, pl.ds(CHUNK, CHUNK)]],
                ssem1).wait()
            pltpu.async_copy(h_hbm.at[combo_v.at[k + 3, pl.ds(0, CHUNK)]],
                             rows1, sem1)

        klast = CHUNKS_PER_WORKER - 2
        pltpu.make_async_copy(h_hbm.at[combo_v.at[klast, pl.ds(0, CHUNK)]],
                              rows0, sem0).wait()
        pltpu.sync_copy(rows0,
                        pooled_sh.at[combo_v.at[klast, pl.ds(CHUNK, CHUNK)]],
                        add=True)
        pltpu.make_async_copy(h_hbm.at[combo_v.at[klast + 1, pl.ds(0, CHUNK)]],
                              rows1, sem1).wait()
        pltpu.sync_copy(rows1,
                        pooled_sh.at[combo_v.at[klast + 1,
                                                pl.ds(CHUNK, CHUNK)]],
                        add=True)

        plsc.subcore_barrier()

        pltpu.sync_copy(pooled_sh.at[pl.ds(row0, ROWS_PER_SUB)],
                        out_hbm.at[pl.ds(c * N + row0, ROWS_PER_SUB)])

        @pl.when(s == NS - 1)
        def _():
            pltpu.sync_copy(pooled_sh.at[pl.ds(TAIL_ROW0, N - TAIL_ROW0)],
                            out_hbm.at[pl.ds(c * N + TAIL_ROW0,
                                             N - TAIL_ROW0)])

    return spmm_kernel(h, idx2, zeros)


def _mlp_body(part_ref, h_ref, scale_ref, w1_ref, b1_ref, g1_ref, be1_ref,
              w2_ref, b2_ref, g2_ref, be2_ref, out_ref):
    pooled = part_ref[0] + part_ref[1] + scale_ref[0, 0] * h_ref[...]
    z = jnp.dot(pooled, w1_ref[...], preferred_element_type=jnp.float32)
    z = z + b1_ref[...]
    mean = jnp.mean(z, axis=0, keepdims=True)
    var = jnp.mean((z - mean) ** 2, axis=0, keepdims=True)
    z = (z - mean) * lax.rsqrt(var + 1e-5) * g1_ref[...] + be1_ref[...]
    z = jnp.maximum(z, 0.0)
    z = jnp.dot(z, w2_ref[...], preferred_element_type=jnp.float32)
    z = z + b2_ref[...]
    mean = jnp.mean(z, axis=0, keepdims=True)
    var = jnp.mean((z - mean) ** 2, axis=0, keepdims=True)
    z = (z - mean) * lax.rsqrt(var + 1e-5) * g2_ref[...] + be2_ref[...]
    out_ref[...] = jnp.maximum(z, 0.0)


def _mlp_tc(parts, h, scale, w1, b1, g1, be1, w2, b2, g2, be2):
    vmem = pl.BlockSpec(memory_space=pltpu.VMEM)
    smem = pl.BlockSpec(memory_space=pltpu.SMEM)
    return pl.pallas_call(
        _mlp_body,
        out_shape=jax.ShapeDtypeStruct((N, D), jnp.float32),
        in_specs=[vmem, vmem, smem, vmem, vmem, vmem, vmem,
                  vmem, vmem, vmem, vmem],
        out_specs=vmem,
    )(parts, h, scale, w1, b1, g1, be1, w2, b2, g2, be2)


def _pad_indices(edge_index):
    si = edge_index[0].astype(jnp.int32).reshape(NW, EDGES_PER_WORKER)
    di = edge_index[1].astype(jnp.int32).reshape(NW, EDGES_PER_WORKER)
    pad_pos = jnp.arange(PAD, dtype=jnp.int32)
    pad_s = jnp.broadcast_to((pad_pos * 37) % N, (NW, PAD))
    pad_d = jnp.broadcast_to(N + pad_pos % DUMP, (NW, PAD))
    s3 = jnp.concatenate([si, pad_s], 1).reshape(NW, CHUNKS_PER_WORKER, CHUNK)
    d3 = jnp.concatenate([di, pad_d], 1).reshape(NW, CHUNKS_PER_WORKER, CHUNK)
    return jnp.concatenate([s3, d3], 2).reshape(-1, 2 * CHUNK)


def kernel(x, edge_index, eps, W1, b1, bn1_g, bn1_b, W2, b2, bn2_g, bn2_b):
    idx2 = _pad_indices(edge_index)
    zeros = jnp.zeros((N + DUMP, D), jnp.float32)
    h = x
    for l in range(L):
        parts = _spmm_sc(h, idx2, zeros).reshape(NC, N, D)
        scale = (1.0 + eps[l]).reshape(1, 1)
        h = _mlp_tc(parts, h, scale,
                    W1[l], b1[l].reshape(1, D),
                    bn1_g[l].reshape(1, D), bn1_b[l].reshape(1, D),
                    W2[l], b2[l].reshape(1, D),
                    bn2_g[l].reshape(1, D), bn2_b[l].reshape(1, D))
    return h


# speedup vs baseline: 1.5121x; 1.5121x over previous
import functools

import jax
import jax.numpy as jnp
from jax import lax
from jax.experimental import pallas as pl
from jax.experimental.pallas import tpu as pltpu
from jax.experimental.pallas import tpu_sc as plsc

N = 10000
E = 320000
D = 128
L = 3

NC = 2
NS = 16
NW = NC * NS
EDGES_PER_WORKER = E // NW
CHUNK = 64
CHUNKS_PER_WORKER = 160
PAD = CHUNKS_PER_WORKER * CHUNK - EDGES_PER_WORKER
DUMP = 8
ROWS_PER_SUB = 624
TAIL_ROW0 = 16 * ROWS_PER_SUB
TAIL_ROWS = N + DUMP - TAIL_ROW0


def _spmm_sc(h, idx2, zeros):
    mesh = plsc.VectorSubcoreMesh(core_axis_name="c", subcore_axis_name="s")

    @functools.partial(
        pl.kernel,
        out_type=jax.ShapeDtypeStruct((NC * N, D), jnp.float32),
        mesh=mesh,
        scratch_types=[
            pltpu.VMEM((CHUNKS_PER_WORKER, 2 * CHUNK), jnp.int32),
            pltpu.VMEM((CHUNK, D), jnp.float32),
            pltpu.VMEM((CHUNK, D), jnp.float32),
            pltpu.VMEM((CHUNK, D), jnp.float32),
            pltpu.VMEM_SHARED((N + DUMP, D), jnp.float32),
            pltpu.SemaphoreType.DMA,
            pltpu.SemaphoreType.DMA,
            pltpu.SemaphoreType.DMA,
            pltpu.SemaphoreType.DMA,
        ],
    )
    def spmm_kernel(h_hbm, idx_hbm, z_hbm, out_hbm,
                    combo_v, rows0, rows1, rows2, pooled_sh,
                    sem0, sem1, sem2, zsem):
        c = lax.axis_index("c")
        s = lax.axis_index("s")
        wid = s * NC + c

        row0 = s * ROWS_PER_SUB
        zcp = pltpu.async_copy(z_hbm.at[pl.ds(row0, ROWS_PER_SUB)],
                               pooled_sh.at[pl.ds(row0, ROWS_PER_SUB)], zsem)

        @pl.when(s == NS - 1)
        def _():
            pltpu.async_copy(z_hbm.at[pl.ds(TAIL_ROW0, TAIL_ROWS)],
                             pooled_sh.at[pl.ds(TAIL_ROW0, TAIL_ROWS)],
                             zsem).wait()

        cbase = wid * CHUNKS_PER_WORKER
        pltpu.sync_copy(idx_hbm.at[pl.ds(cbase, CHUNKS_PER_WORKER)], combo_v)
        zcp.wait()
        plsc.subcore_barrier()

        def gath(k, buf, sem):
            pltpu.async_copy(h_hbm.at[combo_v.at[k, pl.ds(0, CHUNK)]],
                             buf, sem)

        def gwait(k, buf, sem):
            pltpu.make_async_copy(h_hbm.at[combo_v.at[k, pl.ds(0, CHUNK)]],
                                  buf, sem).wait()

        def scat(k, buf):
            pltpu.sync_copy(buf,
                            pooled_sh.at[combo_v.at[k, pl.ds(CHUNK, CHUNK)]],
                            add=True)

        gath(0, rows0, sem0)
        gath(1, rows1, sem1)
        gath(2, rows2, sem2)

        @pl.loop(0, CHUNKS_PER_WORKER - 4, step=3)
        def _(k):
            gwait(k, rows0, sem0)
            scat(k, rows0)
            gath(k + 3, rows0, sem0)
            gwait(k + 1, rows1, sem1)
            scat(k + 1, rows1)
            gath(k + 4, rows1, sem1)
            gwait(k + 2, rows2, sem2)
            scat(k + 2, rows2)
            gath(k + 5, rows2, sem2)

        kl = CHUNKS_PER_WORKER - 4
        gwait(kl, rows0, sem0)
        scat(kl, rows0)
        gath(kl + 3, rows0, sem0)
        gwait(kl + 1, rows1, sem1)
        scat(kl + 1, rows1)
        gwait(kl + 2, rows2, sem2)
        scat(kl + 2, rows2)
        gwait(kl + 3, rows0, sem0)
        scat(kl + 3, rows0)

        plsc.subcore_barrier()

        pltpu.sync_copy(pooled_sh.at[pl.ds(row0, ROWS_PER_SUB)],
                        out_hbm.at[pl.ds(c * N + row0, ROWS_PER_SUB)])

        @pl.when(s == NS - 1)
        def _():
            pltpu.sync_copy(pooled_sh.at[pl.ds(TAIL_ROW0, N - TAIL_ROW0)],
                            out_hbm.at[pl.ds(c * N + TAIL_ROW0,
                                             N - TAIL_ROW0)])

    return spmm_kernel(h, idx2, zeros)


def _mlp_body(part_ref, h_ref, scale_ref, w1_ref, b1_ref, g1_ref, be1_ref,
              w2_ref, b2_ref, g2_ref, be2_ref, out_ref):
    pooled = part_ref[0] + part_ref[1] + scale_ref[0, 0] * h_ref[...]
    z = jnp.dot(pooled, w1_ref[...], preferred_element_type=jnp.float32)
    z = z + b1_ref[...]
    mean = jnp.mean(z, axis=0, keepdims=True)
    var = jnp.mean((z - mean) ** 2, axis=0, keepdims=True)
    z = (z - mean) * lax.rsqrt(var + 1e-5) * g1_ref[...] + be1_ref[...]
    z = jnp.maximum(z, 0.0)
    z = jnp.dot(z, w2_ref[...], preferred_element_type=jnp.float32)
    z = z + b2_ref[...]
    mean = jnp.mean(z, axis=0, keepdims=True)
    var = jnp.mean((z - mean) ** 2, axis=0, keepdims=True)
    z = (z - mean) * lax.rsqrt(var + 1e-5) * g2_ref[...] + be2_ref[...]
    out_ref[...] = jnp.maximum(z, 0.0)


def _mlp_tc(parts, h, scale, w1, b1, g1, be1, w2, b2, g2, be2):
    vmem = pl.BlockSpec(memory_space=pltpu.VMEM)
    smem = pl.BlockSpec(memory_space=pltpu.SMEM)
    return pl.pallas_call(
        _mlp_body,
        out_shape=jax.ShapeDtypeStruct((N, D), jnp.float32),
        in_specs=[vmem, vmem, smem, vmem, vmem, vmem, vmem,
                  vmem, vmem, vmem, vmem],
        out_specs=vmem,
    )(parts, h, scale, w1, b1, g1, be1, w2, b2, g2, be2)


def _pad_indices(edge_index):
    si = edge_index[0].astype(jnp.int32).reshape(NW, EDGES_PER_WORKER)
    di = edge_index[1].astype(jnp.int32).reshape(NW, EDGES_PER_WORKER)
    pad_pos = jnp.arange(PAD, dtype=jnp.int32)
    pad_s = jnp.broadcast_to((pad_pos * 37) % N, (NW, PAD))
    pad_d = jnp.broadcast_to(N + pad_pos % DUMP, (NW, PAD))
    s3 = jnp.concatenate([si, pad_s], 1).reshape(NW, CHUNKS_PER_WORKER, CHUNK)
    d3 = jnp.concatenate([di, pad_d], 1).reshape(NW, CHUNKS_PER_WORKER, CHUNK)
    return jnp.concatenate([s3, d3], 2).reshape(-1, 2 * CHUNK)


def kernel(x, edge_index, eps, W1, b1, bn1_g, bn1_b, W2, b2, bn2_g, bn2_b):
    idx2 = _pad_indices(edge_index)
    zeros = jnp.zeros((N + DUMP, D), jnp.float32)
    h = x
    for l in range(L):
        parts = _spmm_sc(h, idx2, zeros).reshape(NC, N, D)
        scale = (1.0 + eps[l]).reshape(1, 1)
        h = _mlp_tc(parts, h, scale,
                    W1[l], b1[l].reshape(1, D),
                    bn1_g[l].reshape(1, D), bn1_b[l].reshape(1, D),
                    W2[l], b2[l].reshape(1, D),
                    bn2_g[l].reshape(1, D), bn2_b[l].reshape(1, D))
    return h
